# baseline (speedup 1.0000x reference)
import functools

import jax
import jax.numpy as jnp
from jax import lax
from jax.experimental import pallas as pl
from jax.experimental.pallas import tpu as pltpu
from jax.experimental.pallas import tpu_sc as plsc

B, P, F = 4, 8192, 768
NC, NS = 2, 16
NW = NC * NS
WPB = NW // B
RPW = P // WPB
CHUNK = 80
_SIZES = [CHUNK] * 12 + [64]
_OFFS = [CHUNK * i for i in range(12)] + [CHUNK * 12]
NCHUNK = len(_SIZES)

_mesh = plsc.VectorSubcoreMesh(core_axis_name="c", subcore_axis_name="s")


@functools.partial(
    pl.kernel,
    out_type=jax.ShapeDtypeStruct((B, P, F), jnp.float32),
    mesh=_mesh,
    scratch_types=[
        pltpu.VMEM((RPW,), jnp.int32),
        pltpu.VMEM((2, CHUNK, F), jnp.float32),
        pltpu.SemaphoreType.DMA,
    ],
)
def _gather_kernel(x_hbm, idx_hbm, out_hbm, idx_v, rows_v, gsem):
    wid = lax.axis_index("s") * NC + lax.axis_index("c")
    bb = wid // WPB
    pbase = (wid % WPB) * RPW

    pltpu.sync_copy(idx_hbm.at[pl.ds(pbase, RPW)], idx_v)

    def fire_gather(c, b):
        pltpu.async_copy(
            x_hbm.at[bb].at[idx_v.at[pl.ds(_OFFS[c], _SIZES[c])]],
            rows_v.at[b].at[pl.ds(0, _SIZES[c])],
            gsem,
        )

    fire_gather(0, 0)
    fire_gather(1, 1)

    for c in range(NCHUNK):
        b = c % 2
        pltpu.make_async_copy(
            x_hbm.at[bb].at[idx_v.at[pl.ds(0, _SIZES[c])]],
            rows_v.at[b].at[pl.ds(0, _SIZES[c])],
            gsem,
        ).wait()
        pltpu.sync_copy(
            rows_v.at[b].at[pl.ds(0, _SIZES[c])],
            out_hbm.at[bb].at[pl.ds(pbase + _OFFS[c], _SIZES[c])],
        )
        if c + 2 < NCHUNK:
            fire_gather(c + 2, b)


def kernel(x, routing_map):
    return _gather_kernel(x, routing_map)


# speedup vs baseline: 1.0232x; 1.0232x over previous
import functools

import jax
import jax.numpy as jnp
from jax import lax
from jax.experimental import pallas as pl
from jax.experimental.pallas import tpu as pltpu
from jax.experimental.pallas import tpu_sc as plsc

B, P, F = 4, 8192, 768
NC, NS = 2, 16
NW = NC * NS
WPB = NW // B
RPW = P // WPB
CHUNK = 64
NCHUNK = RPW // CHUNK

_mesh = plsc.VectorSubcoreMesh(core_axis_name="c", subcore_axis_name="s")


@functools.partial(
    pl.kernel,
    out_type=jax.ShapeDtypeStruct((B, P, F), jnp.float32),
    mesh=_mesh,
    scratch_types=[
        pltpu.VMEM((RPW,), jnp.int32),
        pltpu.VMEM((2, CHUNK, F), jnp.float32),
        pltpu.SemaphoreType.DMA,
    ],
)
def _gather_kernel(x_hbm, idx_hbm, out_hbm, idx_v, rows_v, gsem):
    wid = lax.axis_index("s") * NC + lax.axis_index("c")
    bb = wid // WPB
    pbase = (wid % WPB) * RPW

    pltpu.sync_copy(idx_hbm.at[pl.ds(pbase, RPW)], idx_v)

    def fire_gather(c, b):
        pltpu.async_copy(
            x_hbm.at[bb].at[idx_v.at[pl.ds(c * CHUNK, CHUNK)]],
            rows_v.at[b],
            gsem,
        )

    fire_gather(0, 0)
    fire_gather(1, 1)

    @pl.loop(0, NCHUNK, step=2)
    def _(k):
        for b in range(2):
            c = k + b
            pltpu.make_async_copy(
                x_hbm.at[bb].at[idx_v.at[pl.ds(0, CHUNK)]], rows_v.at[b], gsem
            ).wait()
            pltpu.sync_copy(
                rows_v.at[b], out_hbm.at[bb].at[pl.ds(pbase + c * CHUNK, CHUNK)]
            )

            @pl.when(c + 2 < NCHUNK)
            def _():
                fire_gather(c + 2, b)


def kernel(x, routing_map):
    return _gather_kernel(x, routing_map)
